# baseline (speedup 1.0000x reference)
import jax
import jax.numpy as jnp
from jax.experimental import pallas as pl
from jax.experimental.pallas import tpu as pltpu

EPS = 1e-8
NTAGS = 8
SQRT_TINV = 3.798282186859221
NEG_BIG = -1e30

BM = 512


def _row_block_kernel(ni, x_ref, tags_ref, out_ref, xn_s):
    s = pl.program_id(0)
    n = xn_s.shape[0]

    @pl.when(s < ni)
    def _normalize_chunk():
        x = x_ref[...]
        norm = jnp.sqrt(jnp.sum(x * x, axis=1, keepdims=True))
        scale = SQRT_TINV / jnp.maximum(norm, EPS)
        xn_s[pl.ds(s * BM, BM), :] = (x * scale).astype(jnp.bfloat16)

    @pl.when(s >= ni)
    def _compute_block():
        i = s - ni
        xi = xn_s[pl.ds(i * BM, BM), :]
        sim = jax.lax.dot_general(
            xi, xn_s[...], (((1,), (1,)), ((), ())),
            preferred_element_type=jnp.float32)
        row_g = jax.lax.broadcasted_iota(jnp.int32, (BM, n), 0) + i * BM
        col_g = jax.lax.broadcasted_iota(jnp.int32, (BM, n), 1)
        e_bf = jnp.exp2(jnp.where(row_g == col_g, NEG_BIG, sim)
                        ).astype(jnp.bfloat16)

        ct = tags_ref[0, :]
        onehot = (ct[:, None] ==
                  jax.lax.broadcasted_iota(jnp.int32, (n, NTAGS), 1)
                  ).astype(jnp.bfloat16)
        r = jax.lax.dot_general(
            e_bf, onehot, (((1,), (0,)), ((), ())),
            preferred_element_type=jnp.float32)

        rt = tags_ref[0, pl.ds(i * BM, BM)]
        sel = (rt[:, None] ==
               jax.lax.broadcasted_iota(jnp.int32, (BM, NTAGS), 1))
        den = jnp.sum(r, axis=1, keepdims=True)
        num = jnp.sum(jnp.where(sel, r, 0.0), axis=1, keepdims=True)
        valid = num > 0.0
        num_safe = jnp.where(valid, num, 1.0)
        den_safe = jnp.where(den > 0.0, den, 1.0)
        losses = -jnp.log(num_safe / den_safe)
        out_ref[0, 0, 0] = jnp.sum(jnp.where(valid, losses, 0.0))
        out_ref[0, 0, 1] = jnp.sum(valid.astype(jnp.float32))


def _final_kernel(p_ref, out_ref):
    nb = p_ref.shape[0]
    loss = p_ref[0, 0, 0]
    cnt = p_ref[0, 0, 1]
    for k in range(1, nb):
        loss += p_ref[k, 0, 0]
        cnt += p_ref[k, 0, 1]
    out_ref[0, 0] = loss / jnp.maximum(cnt, 1.0)


def kernel(representations, ne_tags):
    n, d = representations.shape
    tags = ne_tags.astype(jnp.int32).reshape(1, n)
    ni = n // BM

    def body(*refs):
        _row_block_kernel(ni, *refs)

    partials = pl.pallas_call(
        body,
        grid=(2 * ni,),
        in_specs=[
            pl.BlockSpec((BM, d), lambda s: (jnp.minimum(s, ni - 1), 0)),
            pl.BlockSpec((1, n), lambda s: (0, 0)),
        ],
        out_specs=pl.BlockSpec(
            (1, 1, 2), lambda s: (jnp.maximum(s - ni, 0), 0, 0),
            memory_space=pltpu.SMEM),
        out_shape=jax.ShapeDtypeStruct((ni, 1, 2), jnp.float32),
        scratch_shapes=[
            pltpu.VMEM((n, d), jnp.bfloat16),
        ],
        compiler_params=pltpu.CompilerParams(
            dimension_semantics=("arbitrary",)),
    )(representations, tags)

    out = pl.pallas_call(
        _final_kernel,
        in_specs=[pl.BlockSpec(memory_space=pltpu.SMEM)],
        out_specs=pl.BlockSpec(memory_space=pltpu.SMEM),
        out_shape=jax.ShapeDtypeStruct((1, 1), jnp.float32),
    )(partials)
    return out[0, 0]


# speedup vs baseline: 1.0874x; 1.0874x over previous
import jax
import jax.numpy as jnp
from jax.experimental import pallas as pl
from jax.experimental.pallas import tpu as pltpu

EPS = 1e-8
NTAGS = 8
SQRT_TINV = 3.798282186859221
NEG_BIG = -1e30

BM = 2048


def _row_block_kernel(ni, x_ref, tags_ref, out_ref, xn_s):
    s = pl.program_id(0)
    n = xn_s.shape[0]

    @pl.when(s < ni)
    def _normalize_chunk():
        x = x_ref[...]
        norm = jnp.sqrt(jnp.sum(x * x, axis=1, keepdims=True))
        scale = SQRT_TINV / jnp.maximum(norm, EPS)
        xn_s[pl.ds(s * BM, BM), :] = (x * scale).astype(jnp.bfloat16)

    @pl.when(s >= ni)
    def _compute_block():
        i = s - ni
        xi = xn_s[pl.ds(i * BM, BM), :]
        sim = jax.lax.dot_general(
            xi, xn_s[...], (((1,), (1,)), ((), ())),
            preferred_element_type=jnp.float32)
        row_g = jax.lax.broadcasted_iota(jnp.int32, (BM, n), 0) + i * BM
        col_g = jax.lax.broadcasted_iota(jnp.int32, (BM, n), 1)
        e_bf = jnp.exp2(jnp.where(row_g == col_g, NEG_BIG, sim)
                        ).astype(jnp.bfloat16)

        ct = tags_ref[0, :]
        onehot = (ct[:, None] ==
                  jax.lax.broadcasted_iota(jnp.int32, (n, NTAGS), 1)
                  ).astype(jnp.bfloat16)
        r = jax.lax.dot_general(
            e_bf, onehot, (((1,), (0,)), ((), ())),
            preferred_element_type=jnp.float32)

        rt = tags_ref[0, pl.ds(i * BM, BM)]
        sel = (rt[:, None] ==
               jax.lax.broadcasted_iota(jnp.int32, (BM, NTAGS), 1))
        den = jnp.sum(r, axis=1, keepdims=True)
        num = jnp.sum(jnp.where(sel, r, 0.0), axis=1, keepdims=True)
        valid = num > 0.0
        num_safe = jnp.where(valid, num, 1.0)
        den_safe = jnp.where(den > 0.0, den, 1.0)
        losses = -jnp.log(num_safe / den_safe)
        out_ref[0, 0, 0] = jnp.sum(jnp.where(valid, losses, 0.0))
        out_ref[0, 0, 1] = jnp.sum(valid.astype(jnp.float32))


def _final_kernel(p_ref, out_ref):
    nb = p_ref.shape[0]
    loss = p_ref[0, 0, 0]
    cnt = p_ref[0, 0, 1]
    for k in range(1, nb):
        loss += p_ref[k, 0, 0]
        cnt += p_ref[k, 0, 1]
    out_ref[0, 0] = loss / jnp.maximum(cnt, 1.0)


def kernel(representations, ne_tags):
    n, d = representations.shape
    tags = ne_tags.astype(jnp.int32).reshape(1, n)
    ni = n // BM

    def body(*refs):
        _row_block_kernel(ni, *refs)

    partials = pl.pallas_call(
        body,
        grid=(2 * ni,),
        in_specs=[
            pl.BlockSpec((BM, d), lambda s: (jnp.minimum(s, ni - 1), 0)),
            pl.BlockSpec((1, n), lambda s: (0, 0)),
        ],
        out_specs=pl.BlockSpec(
            (1, 1, 2), lambda s: (jnp.maximum(s - ni, 0), 0, 0),
            memory_space=pltpu.SMEM),
        out_shape=jax.ShapeDtypeStruct((ni, 1, 2), jnp.float32),
        scratch_shapes=[
            pltpu.VMEM((n, d), jnp.bfloat16),
        ],
        compiler_params=pltpu.CompilerParams(
            dimension_semantics=("arbitrary",)),
    )(representations, tags)

    out = pl.pallas_call(
        _final_kernel,
        in_specs=[pl.BlockSpec(memory_space=pltpu.SMEM)],
        out_specs=pl.BlockSpec(memory_space=pltpu.SMEM),
        out_shape=jax.ShapeDtypeStruct((1, 1), jnp.float32),
    )(partials)
    return out[0, 0]
